# baseline (speedup 1.0000x reference)
import functools

import jax
import jax.numpy as jnp
from jax import lax
from jax.experimental import pallas as pl
from jax.experimental.pallas import tpu as pltpu
from jax.experimental.pallas import tpu_sc as plsc

B = 16
NV, NVP, EV = 500, 512, 8000
NE, NEP, EE = 1000, 1024, 16000
EVP, EEP = 8192, 16384
NC, NS = 2, 16
NW = NC * NS
RV = NVP // NW
RE = NEP // NW
_SENT = 1 << 20

_HI = lax.Precision.HIGHEST


def _dot(a, b):
  return jnp.dot(a, b, precision=_HI, preferred_element_type=jnp.float32)


def _split(x):
  hi = x.astype(jnp.bfloat16)
  lo = (x - hi.astype(jnp.float32)).astype(jnp.bfloat16)
  return hi, lo


def _dotb(a, b):
  return jnp.dot(a, b, preferred_element_type=jnp.float32)


def _dot3(a, b_hi, b_lo):
  a_hi, a_lo = _split(a)
  return _dotb(a_hi, b_hi) + _dotb(a_hi, b_lo) + _dotb(a_lo, b_hi)


def _sc_zero(slab, nwords):
  def body(i, _):
    slab[pl.ds(i * 16, 16)] = jnp.zeros((16,), jnp.float32)
    return 0
  lax.fori_loop(0, nwords // 16, body, 0, unroll=8)


def _sc_stage(src_h, sh, tmp_v, sid, ch):
  pltpu.sync_copy(src_h.at[pl.ds(sid * ch, ch)], tmp_v.at[pl.ds(0, ch)])
  pltpu.sync_copy(tmp_v.at[pl.ds(0, ch)], sh.at[pl.ds(sid * ch, ch)])


def _sc_graph(sh_row, sh_col, out_h, row_v, col_v, slab, wid, e, npad, r):
  base = wid * r
  pltpu.sync_copy(sh_row, row_v)
  pltpu.sync_copy(sh_col, col_v)
  _sc_zero(slab, r * npad)
  lanes = lax.iota(jnp.int32, 16)
  for j16 in range(r // 16):
    didx = (j16 * 16 + lanes) * (npad + 1) + base
    plsc.store_scatter(slab, [didx], jnp.ones((16,), jnp.float32))
  ones = jnp.ones((16,), jnp.float32)

  def body(i, _):
    rr = row_v[pl.ds(i * 16, 16)]
    cc = col_v[pl.ds(i * 16, 16)]
    local = cc - base
    m = (local >= 0) & (local < r)
    idx = jnp.where(m, local * npad + rr, 0)
    plsc.addupdate_scatter(slab, [idx], ones, mask=m)
    return 0

  lax.fori_loop(0, e // 16, body, 0, unroll=4)
  pltpu.sync_copy(slab, out_h.at[pl.ds(base * npad, r * npad)])


def _sc_build_m(vrow, vcol, erow, ecol):
  mesh = plsc.VectorSubcoreMesh(core_axis_name="c", subcore_axis_name="s")

  @functools.partial(
      pl.kernel,
      out_type=[
          jax.ShapeDtypeStruct((NVP * NVP,), jnp.float32),
          jax.ShapeDtypeStruct((NEP * NEP,), jnp.float32),
      ],
      mesh=mesh,
      scratch_types=[
          pltpu.VMEM((EVP,), jnp.int32),
          pltpu.VMEM((EVP,), jnp.int32),
          pltpu.VMEM((EEP,), jnp.int32),
          pltpu.VMEM((EEP,), jnp.int32),
          pltpu.VMEM((RV * NVP,), jnp.float32),
          pltpu.VMEM((RE * NEP,), jnp.float32),
          pltpu.VMEM_SHARED((EVP,), jnp.int32),
          pltpu.VMEM_SHARED((EVP,), jnp.int32),
          pltpu.VMEM_SHARED((EEP,), jnp.int32),
          pltpu.VMEM_SHARED((EEP,), jnp.int32),
      ],
      compiler_params=pltpu.CompilerParams(needs_layout_passes=False),
  )
  def k(vrow_h, vcol_h, erow_h, ecol_h, mv_h, me_h,
        vrow_v, vcol_v, erow_v, ecol_v, slab_v, slab_e,
        sh_vrow, sh_vcol, sh_erow, sh_ecol):
    sid = lax.axis_index("s")
    wid = sid * NC + lax.axis_index("c")
    _sc_stage(vrow_h, sh_vrow, vrow_v, sid, EVP // NS)
    _sc_stage(vcol_h, sh_vcol, vcol_v, sid, EVP // NS)
    _sc_stage(erow_h, sh_erow, erow_v, sid, EEP // NS)
    _sc_stage(ecol_h, sh_ecol, ecol_v, sid, EEP // NS)
    plsc.subcore_barrier()
    _sc_graph(sh_vrow, sh_vcol, mv_h, vrow_v, vcol_v, slab_v, wid, EVP, NVP, RV)
    _sc_graph(sh_erow, sh_ecol, me_h, erow_v, ecol_v, slab_e, wid, EEP, NEP, RE)

  return k(vrow, vcol, erow, ecol)


def _gcn_stack(m_ref, x_ref, h_ref, n, npad, w_refs, b_refs):
  ones_col = jnp.ones((npad, 1), jnp.bfloat16)
  deg = _dotb(m_ref[...], ones_col)
  dinv = lax.rsqrt(deg)
  rmask = (lax.broadcasted_iota(jnp.int32, (npad, 1), 0) < n)
  dinv_m = jnp.where(rmask, dinv, 0.0)
  dm_hi, dm_lo = _split(dinv_m)

  def dgt(a):
    return lax.dot_general(a, m_ref[...],
                           dimension_numbers=(((0,), (0,)), ((), ())),
                           preferred_element_type=jnp.float32)

  w_row = dgt(dm_hi) + dgt(dm_lo)

  for l in range(3):
    w = w_refs[l][...]
    bias = b_refs[l][...]
    cin = w_refs[l].shape[0]
    cout = w_refs[l].shape[1]

    def body(b, _, w=w, bias=bias, cin=cin, cout=cout, l=l):
      rows = pl.ds(b * npad, npad)
      hb = x_ref[rows, :] if l == 0 else h_ref[rows, :cin]
      tb = _dotb(hb, w) * dinv
      ub = _dotb(m_ref[...], tb.astype(jnp.bfloat16))
      gb = ub * dinv + bias
      h_ref[rows, :cout] = jnp.where(gb >= 0, gb, 0.1 * gb).astype(jnp.bfloat16)
      return 0

    lax.fori_loop(0, B, body, 0, unroll=2)

  w4 = w_refs[3][...]
  b4 = b_refs[3][...]
  cin = w_refs[3].shape[0]
  inv_n = 1.0 / n
  embs = []
  for b in range(B):
    rows = pl.ds(b * npad, npad)
    tb = _dotb(h_ref[rows, :cin], w4) * dinv
    embs.append(_dot(w_row, tb) * inv_n + b4)
  return jnp.concatenate(embs, axis=0)


def _tc_body(mv_ref, me_ref, xv_ref, xe_ref, nde_ref, mask_ref,
             vw1, vb1, vw2, vb2, vw3, vb3, vw4, vb4,
             ew1, eb1, ew2, eb2, ew3, eb3, ew4, eb4,
             mw1, mb1, mw2, mb2, mw3, mb3, mw4, mb4,
             ff_ref, out_ref, h_ref):
  emb_v = _gcn_stack(mv_ref, xv_ref, h_ref, NV, NVP,
                     (vw1, vw2, vw3, vw4), (vb1, vb2, vb3, vb4))
  emb_e = _gcn_stack(me_ref, xe_ref, h_ref, NE, NEP,
                     (ew1, ew2, ew3, ew4), (eb1, eb2, eb3, eb4))
  ff = jnp.concatenate([emb_v, emb_e, nde_ref[...]], axis=1)
  ff_ref[...] = ff
  h = jnp.tanh(_dot(ff, mw1[...]) + mb1[...])
  h = jnp.tanh(_dot(h, mw2[...]) + mb2[...])
  h = jnp.tanh(_dot(h, mw3[...]) + mb3[...])
  yp = _dot(h, mw4[...]) + mb4[...]
  yp = jnp.where(mask_ref[...] != 0, -jnp.inf, yp)
  mx = jnp.max(yp, axis=-1, keepdims=True)
  lse = jnp.log(jnp.sum(jnp.exp(yp - mx), axis=-1, keepdims=True)) + mx
  out_ref[...] = yp - lse


def _tc_forward(mv, me, xv, xe, nde, maskf, *params):
  return pl.pallas_call(
      _tc_body,
      out_shape=[
          jax.ShapeDtypeStruct((B, 136), jnp.float32),
          jax.ShapeDtypeStruct((B, 11), jnp.float32),
      ],
      scratch_shapes=[pltpu.VMEM((B * NEP, 150), jnp.bfloat16)],
  )(mv, me, xv, xe, nde, maskf, *params)


def kernel(vertiport_features, vertiport_edge, evtol_features, evtol_edge,
           next_drone_embedding, mask,
           vW1, vb1, vW2, vb2, vW3, vb3, vW4, vb4,
           eW1, eb1, eW2, eb2, eW3, eb3, eW4, eb4,
           mW1, mb1, mW2, mb2, mW3, mb3, mW4, mb4):
  vrow = jnp.pad(vertiport_edge[0, 0].astype(jnp.int32), (0, EVP - EV))
  vcol = jnp.pad(vertiport_edge[0, 1].astype(jnp.int32), (0, EVP - EV),
                 constant_values=_SENT)
  erow = jnp.pad(evtol_edge[0, 0].astype(jnp.int32), (0, EEP - EE))
  ecol = jnp.pad(evtol_edge[0, 1].astype(jnp.int32), (0, EEP - EE),
                 constant_values=_SENT)

  mv_flat, me_flat = _sc_build_m(vrow, vcol, erow, ecol)
  mv = mv_flat.reshape(NVP, NVP).astype(jnp.bfloat16)
  me = me_flat.reshape(NEP, NEP).astype(jnp.bfloat16)

  xv = jnp.pad(vertiport_features.astype(jnp.float32),
               ((0, 0), (0, NVP - NV), (0, 0))).reshape(B * NVP, 4)
  xv = xv.astype(jnp.bfloat16)
  xe = jnp.pad(evtol_features.astype(jnp.float32),
               ((0, 0), (0, NEP - NE), (0, 0))).reshape(B * NEP, 5)
  xe = xe.astype(jnp.bfloat16)
  maskf = mask.astype(jnp.float32)

  def r2(v):
    return v.reshape(1, -1)

  def bf(w):
    return w.astype(jnp.bfloat16)

  params = (bf(vW1), r2(vb1), bf(vW2), r2(vb2), bf(vW3), r2(vb3), bf(vW4), r2(vb4),
            bf(eW1), r2(eb1), bf(eW2), r2(eb2), bf(eW3), r2(eb3), bf(eW4), r2(eb4),
            mW1, r2(mb1), mW2, r2(mb2), mW3, r2(mb3), mW4, r2(mb4))

  ff, out = _tc_forward(mv, me, xv, xe,
                        next_drone_embedding.astype(jnp.float32), maskf,
                        *params)
  return (ff, out)


# speedup vs baseline: 1.0825x; 1.0825x over previous
import functools

import jax
import jax.numpy as jnp
from jax import lax
from jax.experimental import pallas as pl
from jax.experimental.pallas import tpu as pltpu
from jax.experimental.pallas import tpu_sc as plsc

B = 16
NV, NVP, EV = 500, 512, 8000
NE, NEP, EE = 1000, 1024, 16000
EVP, EEP = 8192, 16384
NC, NS = 2, 16
NW = NC * NS
RV = NVP // NW
RE = NEP // NW
_SENT = 1 << 20

_HI = lax.Precision.HIGHEST


def _dot(a, b):
  return jnp.dot(a, b, precision=_HI, preferred_element_type=jnp.float32)


def _split(x):
  hi = x.astype(jnp.bfloat16)
  lo = (x - hi.astype(jnp.float32)).astype(jnp.bfloat16)
  return hi, lo


def _dotb(a, b):
  return jnp.dot(a, b, preferred_element_type=jnp.float32)


def _dot3(a, b_hi, b_lo):
  a_hi, a_lo = _split(a)
  return _dotb(a_hi, b_hi) + _dotb(a_hi, b_lo) + _dotb(a_lo, b_hi)


def _sc_zero(slab, nwords):
  def body(i, _):
    slab[pl.ds(i * 16, 16)] = jnp.zeros((16,), jnp.float32)
    return 0
  lax.fori_loop(0, nwords // 16, body, 0, unroll=8)


def _sc_stage(src_h, sh, tmp_v, sid, ch):
  pltpu.sync_copy(src_h.at[pl.ds(sid * ch, ch)], tmp_v.at[pl.ds(0, ch)])
  pltpu.sync_copy(tmp_v.at[pl.ds(0, ch)], sh.at[pl.ds(sid * ch, ch)])


def _sc_graph(sh_row, sh_col, out_h, row_v, col_v, slab, wid, e, npad, r):
  base = wid * r
  pltpu.sync_copy(sh_row, row_v)
  pltpu.sync_copy(sh_col, col_v)
  _sc_zero(slab, r * npad)
  lanes = lax.iota(jnp.int32, 16)
  for j16 in range(r // 16):
    didx = (j16 * 16 + lanes) * (npad + 1) + base
    plsc.store_scatter(slab, [didx], jnp.ones((16,), jnp.float32))
  ones = jnp.ones((16,), jnp.float32)

  def body(i, _):
    rr = row_v[pl.ds(i * 16, 16)]
    cc = col_v[pl.ds(i * 16, 16)]
    local = cc - base
    m = (local >= 0) & (local < r)
    idx = jnp.where(m, local * npad + rr, 0)
    plsc.addupdate_scatter(slab, [idx], ones, mask=m)
    return 0

  lax.fori_loop(0, e // 16, body, 0, unroll=4)
  pltpu.sync_copy(slab, out_h.at[pl.ds(base * npad, r * npad)])


def _sc_build_m(vrow, vcol, erow, ecol):
  mesh = plsc.VectorSubcoreMesh(core_axis_name="c", subcore_axis_name="s")

  @functools.partial(
      pl.kernel,
      out_type=[
          jax.ShapeDtypeStruct((NVP * NVP,), jnp.float32),
          jax.ShapeDtypeStruct((NEP * NEP,), jnp.float32),
      ],
      mesh=mesh,
      scratch_types=[
          pltpu.VMEM((EVP,), jnp.int32),
          pltpu.VMEM((EVP,), jnp.int32),
          pltpu.VMEM((EEP,), jnp.int32),
          pltpu.VMEM((EEP,), jnp.int32),
          pltpu.VMEM((RV * NVP,), jnp.float32),
          pltpu.VMEM((RE * NEP,), jnp.float32),
          pltpu.VMEM_SHARED((EVP,), jnp.int32),
          pltpu.VMEM_SHARED((EVP,), jnp.int32),
          pltpu.VMEM_SHARED((EEP,), jnp.int32),
          pltpu.VMEM_SHARED((EEP,), jnp.int32),
      ],
      compiler_params=pltpu.CompilerParams(needs_layout_passes=False),
  )
  def k(vrow_h, vcol_h, erow_h, ecol_h, mv_h, me_h,
        vrow_v, vcol_v, erow_v, ecol_v, slab_v, slab_e,
        sh_vrow, sh_vcol, sh_erow, sh_ecol):
    sid = lax.axis_index("s")
    wid = sid * NC + lax.axis_index("c")
    _sc_stage(vrow_h, sh_vrow, vrow_v, sid, EVP // NS)
    _sc_stage(vcol_h, sh_vcol, vcol_v, sid, EVP // NS)
    _sc_stage(erow_h, sh_erow, erow_v, sid, EEP // NS)
    _sc_stage(ecol_h, sh_ecol, ecol_v, sid, EEP // NS)
    plsc.subcore_barrier()
    _sc_graph(sh_vrow, sh_vcol, mv_h, vrow_v, vcol_v, slab_v, wid, EVP, NVP, RV)
    _sc_graph(sh_erow, sh_ecol, me_h, erow_v, ecol_v, slab_e, wid, EEP, NEP, RE)

  return k(vrow, vcol, erow, ecol)


def _gcn_stack(m_ref, x_ref, h_ref, n, npad, w_refs, b_refs):
  ones_col = jnp.ones((npad, 1), jnp.bfloat16)
  deg = _dotb(m_ref[...], ones_col)
  dinv = lax.rsqrt(deg)
  rmask = (lax.broadcasted_iota(jnp.int32, (npad, 1), 0) < n)
  dinv_m = jnp.where(rmask, dinv, 0.0)
  dm_hi, dm_lo = _split(dinv_m)

  def dgt(a):
    return lax.dot_general(a, m_ref[...],
                           dimension_numbers=(((0,), (0,)), ((), ())),
                           preferred_element_type=jnp.float32)

  w_row = dgt(dm_hi) + dgt(dm_lo)

  cin0 = x_ref.shape[1]
  dinv_big = jnp.concatenate([dinv] * B, axis=0)
  h_ref[pl.ds(0, B * npad), :cin0] = (x_ref[...] * dinv_big).astype(jnp.bfloat16)

  def dotbf(a, b):
    return jnp.dot(a, b, preferred_element_type=jnp.float32).astype(jnp.bfloat16)

  for l in range(3):
    w = w_refs[l][...]
    bias = b_refs[l][...]
    cin = w_refs[l].shape[0]
    cout = w_refs[l].shape[1]

    def body(b, _, w=w, bias=bias, cin=cin, cout=cout, l=l):
      rows = pl.ds(b * npad, npad)
      tb = dotbf(h_ref[rows, :cin], w)
      ub = _dotb(m_ref[...], tb)
      gb = ub * dinv + bias
      lk = jnp.where(gb >= 0, gb, 0.1 * gb)
      h_ref[rows, :cout] = (lk * dinv).astype(jnp.bfloat16)
      return 0

    lax.fori_loop(0, B, body, 0)

  w4 = w_refs[3][...]
  b4 = b_refs[3][...]
  cin = w_refs[3].shape[0]
  inv_n = 1.0 / n
  wr_hi, wr_lo = _split(w_row)
  embs = []
  for b in range(B):
    rows = pl.ds(b * npad, npad)
    tb = dotbf(h_ref[rows, :cin], w4)
    e = _dotb(wr_hi, tb) + _dotb(wr_lo, tb)
    embs.append(e * inv_n + b4)
  return jnp.concatenate(embs, axis=0)


def _tc_body(mv_ref, me_ref, xv_ref, xe_ref, nde_ref, mask_ref,
             vw1, vb1, vw2, vb2, vw3, vb3, vw4, vb4,
             ew1, eb1, ew2, eb2, ew3, eb3, ew4, eb4,
             mw1, mb1, mw2, mb2, mw3, mb3, mw4, mb4,
             ff_ref, out_ref, h_ref):
  emb_v = _gcn_stack(mv_ref, xv_ref, h_ref, NV, NVP,
                     (vw1, vw2, vw3, vw4), (vb1, vb2, vb3, vb4))
  emb_e = _gcn_stack(me_ref, xe_ref, h_ref, NE, NEP,
                     (ew1, ew2, ew3, ew4), (eb1, eb2, eb3, eb4))
  ff = jnp.concatenate([emb_v, emb_e, nde_ref[...]], axis=1)
  ff_ref[...] = ff
  h = jnp.tanh(_dot(ff, mw1[...]) + mb1[...])
  h = jnp.tanh(_dot(h, mw2[...]) + mb2[...])
  h = jnp.tanh(_dot(h, mw3[...]) + mb3[...])
  yp = _dot(h, mw4[...]) + mb4[...]
  yp = jnp.where(mask_ref[...] != 0, -jnp.inf, yp)
  mx = jnp.max(yp, axis=-1, keepdims=True)
  lse = jnp.log(jnp.sum(jnp.exp(yp - mx), axis=-1, keepdims=True)) + mx
  out_ref[...] = yp - lse


def _tc_forward(mv, me, xv, xe, nde, maskf, *params):
  return pl.pallas_call(
      _tc_body,
      out_shape=[
          jax.ShapeDtypeStruct((B, 136), jnp.float32),
          jax.ShapeDtypeStruct((B, 11), jnp.float32),
      ],
      scratch_shapes=[pltpu.VMEM((B * NEP, 150), jnp.bfloat16)],
  )(mv, me, xv, xe, nde, maskf, *params)


def kernel(vertiport_features, vertiport_edge, evtol_features, evtol_edge,
           next_drone_embedding, mask,
           vW1, vb1, vW2, vb2, vW3, vb3, vW4, vb4,
           eW1, eb1, eW2, eb2, eW3, eb3, eW4, eb4,
           mW1, mb1, mW2, mb2, mW3, mb3, mW4, mb4):
  vrow = jnp.pad(vertiport_edge[0, 0].astype(jnp.int32), (0, EVP - EV))
  vcol = jnp.pad(vertiport_edge[0, 1].astype(jnp.int32), (0, EVP - EV),
                 constant_values=_SENT)
  erow = jnp.pad(evtol_edge[0, 0].astype(jnp.int32), (0, EEP - EE))
  ecol = jnp.pad(evtol_edge[0, 1].astype(jnp.int32), (0, EEP - EE),
                 constant_values=_SENT)

  mv_flat, me_flat = _sc_build_m(vrow, vcol, erow, ecol)
  mv = mv_flat.reshape(NVP, NVP).astype(jnp.bfloat16)
  me = me_flat.reshape(NEP, NEP).astype(jnp.bfloat16)

  xv = jnp.pad(vertiport_features.astype(jnp.float32),
               ((0, 0), (0, NVP - NV), (0, 0))).reshape(B * NVP, 4)
  xv = xv.astype(jnp.bfloat16)
  xe = jnp.pad(evtol_features.astype(jnp.float32),
               ((0, 0), (0, NEP - NE), (0, 0))).reshape(B * NEP, 5)
  xe = xe.astype(jnp.bfloat16)
  maskf = mask.astype(jnp.float32)

  def r2(v):
    return v.reshape(1, -1)

  def bf(w):
    return w.astype(jnp.bfloat16)

  params = (bf(vW1), r2(vb1), bf(vW2), r2(vb2), bf(vW3), r2(vb3), bf(vW4), r2(vb4),
            bf(eW1), r2(eb1), bf(eW2), r2(eb2), bf(eW3), r2(eb3), bf(eW4), r2(eb4),
            mW1, r2(mb1), mW2, r2(mb2), mW3, r2(mb3), mW4, r2(mb4))

  ff, out = _tc_forward(mv, me, xv, xe,
                        next_drone_embedding.astype(jnp.float32), maskf,
                        *params)
  return (ff, out)


# speedup vs baseline: 1.0829x; 1.0003x over previous
import functools

import jax
import jax.numpy as jnp
from jax import lax
from jax.experimental import pallas as pl
from jax.experimental.pallas import tpu as pltpu
from jax.experimental.pallas import tpu_sc as plsc

B = 16
NV, NVP, EV = 500, 512, 8000
NE, NEP, EE = 1000, 1024, 16000
EVP, EEP = 8192, 16384
NC, NS = 2, 16
NW = NC * NS
RV = NVP // NW
RE = NEP // NW
_SENT = 1 << 20

_HI = lax.Precision.HIGHEST


def _dot(a, b):
  return jnp.dot(a, b, precision=_HI, preferred_element_type=jnp.float32)


def _split(x):
  hi = x.astype(jnp.bfloat16)
  lo = (x - hi.astype(jnp.float32)).astype(jnp.bfloat16)
  return hi, lo


def _dotb(a, b):
  return jnp.dot(a, b, preferred_element_type=jnp.float32)


def _sc_zero(slab, nwords):
  def body(i, _):
    slab[pl.ds(i * 16, 16)] = jnp.zeros((16,), jnp.float32)
    return 0
  lax.fori_loop(0, nwords // 16, body, 0, unroll=8)


def _sc_stage(src_h, sh, tmp_v, sid, ch):
  pltpu.sync_copy(src_h.at[pl.ds(sid * ch, ch)], tmp_v.at[pl.ds(0, ch)])
  pltpu.sync_copy(tmp_v.at[pl.ds(0, ch)], sh.at[pl.ds(sid * ch, ch)])


def _sc_graph(sh_row, sh_col, out_h, row_v, col_v, slab, wid, e, npad, r):
  base = wid * r
  pltpu.sync_copy(sh_row, row_v)
  pltpu.sync_copy(sh_col, col_v)
  _sc_zero(slab, r * npad)
  lanes = lax.iota(jnp.int32, 16)
  for j16 in range(r // 16):
    didx = (j16 * 16 + lanes) * (npad + 1) + base
    plsc.store_scatter(slab, [didx], jnp.ones((16,), jnp.float32))
  ones = jnp.ones((16,), jnp.float32)

  def body(i, _):
    rr = row_v[pl.ds(i * 16, 16)]
    cc = col_v[pl.ds(i * 16, 16)]
    local = cc - base
    m = (local >= 0) & (local < r)
    idx = jnp.where(m, local * npad + rr, 0)
    plsc.addupdate_scatter(slab, [idx], ones, mask=m)
    return 0

  lax.fori_loop(0, e // 16, body, 0, unroll=4)
  pltpu.sync_copy(slab, out_h.at[pl.ds(base * npad, r * npad)])


def _sc_build_m(vrow, vcol, erow, ecol):
  mesh = plsc.VectorSubcoreMesh(core_axis_name="c", subcore_axis_name="s")

  @functools.partial(
      pl.kernel,
      out_type=[
          jax.ShapeDtypeStruct((NVP * NVP,), jnp.float32),
          jax.ShapeDtypeStruct((NEP * NEP,), jnp.float32),
      ],
      mesh=mesh,
      scratch_types=[
          pltpu.VMEM((EVP,), jnp.int32),
          pltpu.VMEM((EVP,), jnp.int32),
          pltpu.VMEM((EEP,), jnp.int32),
          pltpu.VMEM((EEP,), jnp.int32),
          pltpu.VMEM((RV * NVP,), jnp.float32),
          pltpu.VMEM((RE * NEP,), jnp.float32),
          pltpu.VMEM_SHARED((EVP,), jnp.int32),
          pltpu.VMEM_SHARED((EVP,), jnp.int32),
          pltpu.VMEM_SHARED((EEP,), jnp.int32),
          pltpu.VMEM_SHARED((EEP,), jnp.int32),
      ],
      compiler_params=pltpu.CompilerParams(needs_layout_passes=False),
  )
  def k(vrow_h, vcol_h, erow_h, ecol_h, mv_h, me_h,
        vrow_v, vcol_v, erow_v, ecol_v, slab_v, slab_e,
        sh_vrow, sh_vcol, sh_erow, sh_ecol):
    sid = lax.axis_index("s")
    wid = sid * NC + lax.axis_index("c")
    _sc_stage(vrow_h, sh_vrow, vrow_v, sid, EVP // NS)
    _sc_stage(vcol_h, sh_vcol, vcol_v, sid, EVP // NS)
    _sc_stage(erow_h, sh_erow, erow_v, sid, EEP // NS)
    _sc_stage(ecol_h, sh_ecol, ecol_v, sid, EEP // NS)
    plsc.subcore_barrier()
    _sc_graph(sh_vrow, sh_vcol, mv_h, vrow_v, vcol_v, slab_v, wid, EVP, NVP, RV)
    _sc_graph(sh_erow, sh_ecol, me_h, erow_v, ecol_v, slab_e, wid, EEP, NEP, RE)

  return k(vrow, vcol, erow, ecol)


def _gcn_stack(m_ref, x_ref, h_ref, n, npad, w_refs, b_refs):
  ones_col = jnp.ones((npad, 1), jnp.bfloat16)
  deg = _dotb(m_ref[...], ones_col)
  dinv = lax.rsqrt(deg)
  rmask = (lax.broadcasted_iota(jnp.int32, (npad, 1), 0) < n)
  dinv_m = jnp.where(rmask, dinv, 0.0)
  dm_hi, dm_lo = _split(dinv_m)

  def dgt(a):
    return lax.dot_general(a, m_ref[...],
                           dimension_numbers=(((0,), (0,)), ((), ())),
                           preferred_element_type=jnp.float32)

  w_row = dgt(dm_hi) + dgt(dm_lo)

  cin0 = x_ref.shape[1]
  dinv_big = jnp.concatenate([dinv] * B, axis=0)
  h_ref[pl.ds(0, B * npad), :cin0] = (x_ref[...] * dinv_big).astype(jnp.bfloat16)

  def dotbf(a, b):
    return jnp.dot(a, b, preferred_element_type=jnp.float32).astype(jnp.bfloat16)

  for l in range(3):
    w = w_refs[l][...]
    bias = b_refs[l][...]
    cin = w_refs[l].shape[0]
    cout = w_refs[l].shape[1]

    def body(b, _, w=w, bias=bias, cin=cin, cout=cout, l=l):
      rows = pl.ds(b * npad, npad)
      tb = dotbf(h_ref[rows, :cin], w)
      ub = _dotb(m_ref[...], tb)
      gb = ub * dinv + bias
      lk = jnp.where(gb >= 0, gb, 0.1 * gb)
      h_ref[rows, :cout] = (lk * dinv).astype(jnp.bfloat16)
      return 0

    lax.fori_loop(0, B, body, 0)

  w4 = w_refs[3][...]
  b4 = b_refs[3][...]
  cin = w_refs[3].shape[0]
  inv_n = 1.0 / n
  wr_hi, wr_lo = _split(w_row)
  embs = []
  for b in range(B):
    rows = pl.ds(b * npad, npad)
    tb = dotbf(h_ref[rows, :cin], w4)
    e = _dotb(wr_hi, tb) + _dotb(wr_lo, tb)
    embs.append(e * inv_n + b4)
  return jnp.concatenate(embs, axis=0)


def _tc_body(mv_ref, me_ref, xv_ref, xe_ref, nde_ref, mask_ref,
             vw1, vb1, vw2, vb2, vw3, vb3, vw4, vb4,
             ew1, eb1, ew2, eb2, ew3, eb3, ew4, eb4,
             mw1, mb1, mw2, mb2, mw3, mb3, mw4, mb4,
             ff_ref, out_ref, h_ref):
  emb_v = _gcn_stack(mv_ref, xv_ref, h_ref, NV, NVP,
                     (vw1, vw2, vw3, vw4), (vb1, vb2, vb3, vb4))
  emb_e = _gcn_stack(me_ref, xe_ref, h_ref, NE, NEP,
                     (ew1, ew2, ew3, ew4), (eb1, eb2, eb3, eb4))
  ff = jnp.concatenate([emb_v, emb_e, nde_ref[...]], axis=1)
  ff_ref[...] = ff
  h = jnp.tanh(_dot(ff, mw1[...]) + mb1[...])
  h = jnp.tanh(_dot(h, mw2[...]) + mb2[...])
  h = jnp.tanh(_dot(h, mw3[...]) + mb3[...])
  yp = _dot(h, mw4[...]) + mb4[...]
  yp = jnp.where(mask_ref[...] != 0, -jnp.inf, yp)
  mx = jnp.max(yp, axis=-1, keepdims=True)
  lse = jnp.log(jnp.sum(jnp.exp(yp - mx), axis=-1, keepdims=True)) + mx
  out_ref[...] = yp - lse


def _tc_forward(mv, me, xv, xe, nde, maskf, *params):
  return pl.pallas_call(
      _tc_body,
      out_shape=[
          jax.ShapeDtypeStruct((B, 136), jnp.float32),
          jax.ShapeDtypeStruct((B, 11), jnp.float32),
      ],
      scratch_shapes=[pltpu.VMEM((B * NEP, 150), jnp.bfloat16)],
  )(mv, me, xv, xe, nde, maskf, *params)


def kernel(vertiport_features, vertiport_edge, evtol_features, evtol_edge,
           next_drone_embedding, mask,
           vW1, vb1, vW2, vb2, vW3, vb3, vW4, vb4,
           eW1, eb1, eW2, eb2, eW3, eb3, eW4, eb4,
           mW1, mb1, mW2, mb2, mW3, mb3, mW4, mb4):
  vrow = jnp.pad(vertiport_edge[0, 0].astype(jnp.int32), (0, EVP - EV))
  vcol = jnp.pad(vertiport_edge[0, 1].astype(jnp.int32), (0, EVP - EV),
                 constant_values=_SENT)
  erow = jnp.pad(evtol_edge[0, 0].astype(jnp.int32), (0, EEP - EE))
  ecol = jnp.pad(evtol_edge[0, 1].astype(jnp.int32), (0, EEP - EE),
                 constant_values=_SENT)

  mv_flat, me_flat = _sc_build_m(vrow, vcol, erow, ecol)
  mv = mv_flat.reshape(NVP, NVP).astype(jnp.bfloat16)
  me = me_flat.reshape(NEP, NEP).astype(jnp.bfloat16)

  xv = jnp.pad(vertiport_features.astype(jnp.float32),
               ((0, 0), (0, NVP - NV), (0, 0))).reshape(B * NVP, 4)
  xv = xv.astype(jnp.bfloat16)
  xe = jnp.pad(evtol_features.astype(jnp.float32),
               ((0, 0), (0, NEP - NE), (0, 0))).reshape(B * NEP, 5)
  xe = xe.astype(jnp.bfloat16)
  maskf = mask.astype(jnp.float32)

  def r2(v):
    return v.reshape(1, -1)

  def bf(w):
    return w.astype(jnp.bfloat16)

  params = (bf(vW1), r2(vb1), bf(vW2), r2(vb2), bf(vW3), r2(vb3), bf(vW4), r2(vb4),
            bf(eW1), r2(eb1), bf(eW2), r2(eb2), bf(eW3), r2(eb3), bf(eW4), r2(eb4),
            mW1, r2(mb1), mW2, r2(mb2), mW3, r2(mb3), mW4, r2(mb4))

  ff, out = _tc_forward(mv, me, xv, xe,
                        next_drone_embedding.astype(jnp.float32), maskf,
                        *params)
  return (ff, out)
